# baseline (speedup 1.0000x reference)
import jax
import jax.numpy as jnp
from jax import lax
from jax.experimental import pallas as pl
from jax.experimental.pallas import tpu as pltpu
from jax.experimental.pallas import tpu_sc as plsc

B, N, F, E, D = 4, 10000, 128, 320000, 128

NUM_TILES = 16
EDGES_PER_TILE = E // NUM_TILES
KBLK = 80
NBLK = EDGES_PER_TILE // KBLK
ROWS_MAIN = 632
ROWS_LAST = N - 15 * ROWS_MAIN
IDEPTH = 8


def _mm_body(a_ref, w_ref, o_ref):
    o_ref[...] = jnp.dot(a_ref[...], w_ref[...],
                         preferred_element_type=jnp.float32)


def _tc_matmul(a, w):
    bm = 2000
    return pl.pallas_call(
        _mm_body,
        grid=(a.shape[0] // bm,),
        in_specs=[
            pl.BlockSpec((bm, F), lambda i: (i, 0)),
            pl.BlockSpec((F, D), lambda i: (0, 0)),
        ],
        out_specs=pl.BlockSpec((bm, D), lambda i: (i, 0)),
        out_shape=jax.ShapeDtypeStruct((a.shape[0], D), jnp.float32),
    )(a, w)


def _tanh_body(y_ref, o_ref):
    o_ref[...] = jnp.tanh(y_ref[...])


def _tc_tanh(y):
    bm = 2000
    return pl.pallas_call(
        _tanh_body,
        grid=(y.shape[0] // bm,),
        in_specs=[pl.BlockSpec((bm, D), lambda i: (i, 0))],
        out_specs=pl.BlockSpec((bm, D), lambda i: (i, 0)),
        out_shape=jax.ShapeDtypeStruct(y.shape, jnp.float32),
    )(y)


def _sc_body(x_hbm, cr_hbm, vals_hbm, zeros_hbm, y_hbm,
             acc, g0, g1, g2, g3, crb, vv_r, gsem, ssem, isem):
    c = lax.axis_index("c")
    s = lax.axis_index("s")
    ebase = s * EDGES_PER_TILE
    rbase = s * ROWS_MAIN
    gbufs = (g0, g1, g2, g3)

    def rows_slice(fn):
        @pl.when(s < 15)
        def _():
            fn(ROWS_MAIN)

        @pl.when(s == 15)
        def _():
            fn(ROWS_LAST)

    class _Multi:
        def __init__(self, ds):
            self.ds = ds

        def start(self):
            for d_ in self.ds:
                d_.start()

        def wait(self):
            for d_ in self.ds:
                d_.wait()

    def fetch(j):
        slot = lax.rem(j, IDEPTH)
        off = ebase + j * KBLK
        return _Multi([
            pltpu.make_async_copy(cr_hbm.at[s * NBLK + j],
                                  crb.at[slot], isem.at[slot]),
            pltpu.make_async_copy(vals_hbm.at[pl.ds(off, KBLK)],
                                  vv_r.at[slot], isem.at[slot]),
        ])

    @pl.loop(0, 2)
    def _(bi):
        b = 2 * c + bi
        xb = x_hbm.at[b]

        def gather(buf, j, sem):
            return pltpu.make_async_copy(
                xb.at[crb.at[lax.rem(j, IDEPTH), 0, pl.ds(0, KBLK)]],
                buf, sem)

        def scatter(buf, j, sem):
            return pltpu.make_async_copy(
                buf,
                acc.at[crb.at[lax.rem(j, IDEPTH), 0, pl.ds(128, KBLK)]],
                sem)

        rows_slice(lambda nr: pltpu.sync_copy(
            zeros_hbm.at[pl.ds(rbase, nr), :], acc.at[pl.ds(rbase, nr), :]))
        plsc.subcore_barrier()

        for j in range(4):
            fetch(j).start()
        for j in range(2):
            fetch(j).wait()
            gather(gbufs[j], j, gsem.at[j]).start()

        def block_body(jb, p, steady):
            pn = (p + 2) % 4
            gather(gbufs[p], jb, gsem.at[p]).wait()

            if steady:
                @pl.when(jb >= 2)
                def _():
                    scatter(gbufs[pn], jb - 2, ssem.at[pn]).wait()

                @pl.when(jb + 2 < NBLK)
                def _():
                    fetch(jb + 2).wait()
                    gather(gbufs[pn], jb + 2, gsem.at[pn]).start()

                @pl.when(jb + 4 < NBLK)
                def _():
                    fetch(jb + 4).start()
            else:
                scatter(gbufs[pn], jb - 2, ssem.at[pn]).wait()

            slot = lax.rem(jb, IDEPTH)

            @pl.loop(0, KBLK, step=16)
            def _(k0):
                vvec = vv_r[slot, pl.ds(k0, 16)]
                for j in range(16):
                    v = vvec[j]
                    for f in range(D // 16):
                        sl = pl.ds(f * 16, 16)
                        gbufs[p][k0 + j, sl] = gbufs[p][k0 + j, sl] * v

            scatter(gbufs[p], jb, ssem.at[p]).start(add=True)

        @pl.loop(0, NBLK - 2, step=4)
        def _(i):
            for p in range(4):
                block_body(i + p, p, steady=True)

        block_body(NBLK - 2, (NBLK - 2) % 4, steady=False)
        block_body(NBLK - 1, (NBLK - 1) % 4, steady=False)
        scatter(gbufs[(NBLK - 2) % 4], NBLK - 2,
                ssem.at[(NBLK - 2) % 4]).wait()
        scatter(gbufs[(NBLK - 1) % 4], NBLK - 1,
                ssem.at[(NBLK - 1) % 4]).wait()

        plsc.subcore_barrier()
        rows_slice(lambda nr: pltpu.sync_copy(
            acc.at[pl.ds(rbase, nr), :],
            y_hbm.at[b].at[pl.ds(rbase, nr), :]))


@jax.jit
def _sc_aggregate(x, rows, cols, vals):
    mesh = plsc.VectorSubcoreMesh(core_axis_name="c", subcore_axis_name="s")
    kern = pl.kernel(
        _sc_body,
        out_type=jax.ShapeDtypeStruct((B, N, D), jnp.float32),
        mesh=mesh,
        scratch_types=[
            pltpu.VMEM_SHARED((N, D), jnp.float32),
            pltpu.VMEM((KBLK, D), jnp.float32),
            pltpu.VMEM((KBLK, D), jnp.float32),
            pltpu.VMEM((KBLK, D), jnp.float32),
            pltpu.VMEM((KBLK, D), jnp.float32),
            pltpu.VMEM((IDEPTH, 1, 256), jnp.int32),
            pltpu.VMEM((IDEPTH, KBLK), jnp.float32),
            pltpu.SemaphoreType.DMA((4,)),
            pltpu.SemaphoreType.DMA((4,)),
            pltpu.SemaphoreType.DMA((IDEPTH,)),
        ],
    )
    zeros = jnp.zeros((N, D), jnp.float32)
    pad = ((0, 0), (0, 128 - KBLK))
    cr = jnp.stack([jnp.pad(cols.reshape(-1, KBLK), pad),
                    jnp.pad(rows.reshape(-1, KBLK), pad)],
                   axis=1).reshape(-1, 1, 256)
    return kern(x, cr, vals, zeros)


def kernel(adj_indices, adj_values, inputs, W):
    b, n, f = inputs.shape
    d = W.shape[1]
    rows = adj_indices[0].astype(jnp.int32)
    cols = adj_indices[1].astype(jnp.int32)
    x = _tc_matmul(inputs.reshape(b * n, f), W).reshape(b, n, d)
    y = _sc_aggregate(x, rows, cols, adj_values)
    out = _tc_tanh(y.reshape(b * n, d))
    return out.reshape(b, n, d)


# speedup vs baseline: 1.0617x; 1.0617x over previous
import jax
import jax.numpy as jnp
from jax import lax
from jax.experimental import pallas as pl
from jax.experimental.pallas import tpu as pltpu
from jax.experimental.pallas import tpu_sc as plsc

B, N, F, E, D = 4, 10000, 128, 320000, 128

NUM_TILES = 16
EDGES_PER_TILE = E // NUM_TILES
KBLK = 80
NBLK = EDGES_PER_TILE // KBLK
ROWS_MAIN = 632
ROWS_LAST = N - 15 * ROWS_MAIN
IDEPTH = 8


def _mm_body(a_ref, w_ref, o_ref):
    o_ref[...] = jnp.dot(a_ref[...], w_ref[...],
                         preferred_element_type=jnp.float32)


def _tc_matmul(a, w):
    bm = 5000
    return pl.pallas_call(
        _mm_body,
        grid=(a.shape[0] // bm,),
        in_specs=[
            pl.BlockSpec((bm, F), lambda i: (i, 0)),
            pl.BlockSpec((F, D), lambda i: (0, 0)),
        ],
        out_specs=pl.BlockSpec((bm, D), lambda i: (i, 0)),
        out_shape=jax.ShapeDtypeStruct((a.shape[0], D), jnp.float32),
    )(a, w)


def _tanh_body(y_ref, o_ref):
    o_ref[...] = jnp.tanh(y_ref[...])


def _tc_tanh(y):
    bm = 5000
    return pl.pallas_call(
        _tanh_body,
        grid=(y.shape[0] // bm,),
        in_specs=[pl.BlockSpec((bm, D), lambda i: (i, 0))],
        out_specs=pl.BlockSpec((bm, D), lambda i: (i, 0)),
        out_shape=jax.ShapeDtypeStruct(y.shape, jnp.float32),
    )(y)


def _sc_body(x_hbm, rows_hbm, cols_hbm, vals_hbm, zeros_hbm, y_hbm,
             acc, g0, g1, g2, g3, cv, rv, vv_r, gsem, ssem, isem):
    c = lax.axis_index("c")
    s = lax.axis_index("s")
    ebase = s * EDGES_PER_TILE
    rbase = s * ROWS_MAIN
    gbufs = (g0, g1, g2, g3)

    def rows_slice(fn):
        @pl.when(s < 15)
        def _():
            fn(ROWS_MAIN)

        @pl.when(s == 15)
        def _():
            fn(ROWS_LAST)

    class _Multi:
        def __init__(self, ds):
            self.ds = ds

        def start(self):
            for d_ in self.ds:
                d_.start()

        def wait(self):
            for d_ in self.ds:
                d_.wait()

    def fetch(j):
        slot = lax.rem(j, IDEPTH)
        off = ebase + j * KBLK
        return _Multi([
            pltpu.make_async_copy(cols_hbm.at[pl.ds(off, KBLK)],
                                  cv.at[slot], isem.at[slot]),
            pltpu.make_async_copy(rows_hbm.at[pl.ds(off, KBLK)],
                                  rv.at[slot], isem.at[slot]),
            pltpu.make_async_copy(vals_hbm.at[pl.ds(off, KBLK)],
                                  vv_r.at[slot], isem.at[slot]),
        ])

    @pl.loop(0, 2)
    def _(bi):
        b = 2 * c + bi
        xb = x_hbm.at[b]

        def gather(buf, j, sem):
            return pltpu.make_async_copy(
                xb.at[cv.at[lax.rem(j, IDEPTH)]], buf, sem)

        def scatter(buf, j, sem):
            return pltpu.make_async_copy(
                buf, acc.at[rv.at[lax.rem(j, IDEPTH)]], sem)

        rows_slice(lambda nr: pltpu.sync_copy(
            zeros_hbm.at[pl.ds(rbase, nr), :], acc.at[pl.ds(rbase, nr), :]))
        plsc.subcore_barrier()

        for j in range(4):
            fetch(j).start()
        for j in range(2):
            fetch(j).wait()
            gather(gbufs[j], j, gsem.at[j]).start()

        def block_body(jb, p, steady):
            pn = (p + 2) % 4
            gather(gbufs[p], jb, gsem.at[p]).wait()

            if steady:
                @pl.when(jb >= 2)
                def _():
                    scatter(gbufs[pn], jb - 2, ssem.at[pn]).wait()

                @pl.when(jb + 2 < NBLK)
                def _():
                    fetch(jb + 2).wait()
                    gather(gbufs[pn], jb + 2, gsem.at[pn]).start()

                @pl.when(jb + 4 < NBLK)
                def _():
                    fetch(jb + 4).start()
            else:
                scatter(gbufs[pn], jb - 2, ssem.at[pn]).wait()

            slot = lax.rem(jb, IDEPTH)

            @pl.loop(0, KBLK, step=16)
            def _(k0):
                vvec = vv_r[slot, pl.ds(k0, 16)]
                for j in range(16):
                    v = vvec[j]
                    for f in range(D // 16):
                        sl = pl.ds(f * 16, 16)
                        gbufs[p][k0 + j, sl] = gbufs[p][k0 + j, sl] * v

            scatter(gbufs[p], jb, ssem.at[p]).start(add=True)

        @pl.loop(0, NBLK - 2, step=4)
        def _(i):
            for p in range(4):
                block_body(i + p, p, steady=True)

        block_body(NBLK - 2, (NBLK - 2) % 4, steady=False)
        block_body(NBLK - 1, (NBLK - 1) % 4, steady=False)
        scatter(gbufs[(NBLK - 2) % 4], NBLK - 2,
                ssem.at[(NBLK - 2) % 4]).wait()
        scatter(gbufs[(NBLK - 1) % 4], NBLK - 1,
                ssem.at[(NBLK - 1) % 4]).wait()

        plsc.subcore_barrier()
        rows_slice(lambda nr: pltpu.sync_copy(
            acc.at[pl.ds(rbase, nr), :],
            y_hbm.at[b].at[pl.ds(rbase, nr), :]))


@jax.jit
def _sc_aggregate(x, rows, cols, vals):
    mesh = plsc.VectorSubcoreMesh(core_axis_name="c", subcore_axis_name="s")
    kern = pl.kernel(
        _sc_body,
        out_type=jax.ShapeDtypeStruct((B, N, D), jnp.float32),
        mesh=mesh,
        scratch_types=[
            pltpu.VMEM_SHARED((N, D), jnp.float32),
            pltpu.VMEM((KBLK, D), jnp.float32),
            pltpu.VMEM((KBLK, D), jnp.float32),
            pltpu.VMEM((KBLK, D), jnp.float32),
            pltpu.VMEM((KBLK, D), jnp.float32),
            pltpu.VMEM((IDEPTH, KBLK), jnp.int32),
            pltpu.VMEM((IDEPTH, KBLK), jnp.int32),
            pltpu.VMEM((IDEPTH, KBLK), jnp.float32),
            pltpu.SemaphoreType.DMA((4,)),
            pltpu.SemaphoreType.DMA((4,)),
            pltpu.SemaphoreType.DMA((IDEPTH,)),
        ],
    )
    zeros = jnp.zeros((N, D), jnp.float32)
    return kern(x, rows, cols, vals, zeros)


def kernel(adj_indices, adj_values, inputs, W):
    b, n, f = inputs.shape
    d = W.shape[1]
    rows = adj_indices[0].astype(jnp.int32)
    cols = adj_indices[1].astype(jnp.int32)
    x = _tc_matmul(inputs.reshape(b * n, f), W).reshape(b, n, d)
    y = _sc_aggregate(x, rows, cols, adj_values)
    out = _tc_tanh(y.reshape(b * n, d))
    return out.reshape(b, n, d)
